# baseline (speedup 1.0000x reference)
import functools

import jax
import jax.numpy as jnp
from jax import lax
from jax.experimental import pallas as pl
from jax.experimental.pallas import tpu as pltpu
from jax.experimental.pallas import tpu_sc as plsc

N = 10000
D = 128
E = 320000

NC = 2
NS = 16
NW = NC * NS
NPAD = 10240
RPT = NPAD // NS
EPT = E // NW
CHUNK = 128
NCH = 80
EPTP = NCH * CHUNK
TRASH = NPAD - 1
DEGW = 16

BR = 1280
GRID = NPAD // BR

_mesh = plsc.VectorSubcoreMesh(core_axis_name="c", subcore_axis_name="s",
                               num_cores=NC, num_subcores=NS)


NB = 2

_AGG_KERNEL_ARGS = dict(
    out_type=jax.ShapeDtypeStruct((NC, NPAD, D), jnp.float32),
    mesh=_mesh,
    scratch_types=[
        pltpu.VMEM((NCH, CHUNK), jnp.int32),
        pltpu.VMEM((NCH, CHUNK), jnp.int32),
        pltpu.VMEM((CHUNK, D), jnp.float32),
        pltpu.VMEM_SHARED((NPAD, D), jnp.float32),
        pltpu.SemaphoreType.DMA,
    ],
)


def _agg_body(msg, srcs, dsts, zd_h, part, si, di, rows, acc, gsem):
    c = lax.axis_index("c")
    s = lax.axis_index("s")
    wid = s * NC + c
    rs = pl.ds(s * RPT, RPT)
    pltpu.sync_copy(zd_h.at[rs], acc.at[rs])
    pltpu.sync_copy(srcs.at[wid], si)
    pltpu.sync_copy(dsts.at[wid], di)
    plsc.subcore_barrier()

    def body(j, carry):
        pltpu.async_copy(msg.at[si.at[j]], rows, gsem).wait()
        pltpu.sync_copy(rows, acc.at[di.at[j]], add=True)
        return carry

    lax.fori_loop(0, NCH, body, 0)
    plsc.subcore_barrier()
    pltpu.sync_copy(acc.at[rs], part.at[c, rs])


_agg_kernel = pl.kernel(_agg_body, **_AGG_KERNEL_ARGS)


def _prep_body(do0, do1, di0, di1, f, w, ns_o, nd_o, msg_o):
    ns = lax.rsqrt(do0[:, 0:1] + do1[:, 0:1] + 1.0)
    nd = lax.rsqrt(di0[:, 0:1] + di1[:, 0:1] + 1.0)
    wide = jnp.ones((1, DEGW), jnp.float32)
    ns_o[...] = ns * wide
    nd_o[...] = nd * wide
    msg_o[...] = jnp.dot(f[...] * ns, w[...],
                         preferred_element_type=jnp.float32)


def _mid_body(p0, p1, msg, nd, ns, b, w, jk_in, jk_o, msg_o):
    agg = p0[...] + p1[...] + msg[...]
    h = jnp.maximum(agg * nd[:, 0:1] + b[...], 0.0)
    jk_o[...] = jnp.maximum(jk_in[...], h)
    msg_o[...] = jnp.dot(h * ns[:, 0:1], w[...],
                         preferred_element_type=jnp.float32)


def _fin_body(p0, p1, msg, nd, b, jk_in, jk_o):
    agg = p0[...] + p1[...] + msg[...]
    h = jnp.maximum(agg * nd[:, 0:1] + b[...], 0.0)
    jk_o[...] = jnp.maximum(jk_in[...], h)


def _rb(width):
    return pl.BlockSpec((BR, width), lambda i: (i, 0))


_FULL_W = pl.BlockSpec((D, D), lambda i: (0, 0))
_FULL_B = pl.BlockSpec((1, D), lambda i: (0, 0))

_prep = pl.pallas_call(
    _prep_body,
    grid=(GRID,),
    in_specs=[_rb(D), _rb(D), _rb(D), _rb(D), _rb(D), _FULL_W],
    out_specs=[_rb(DEGW), _rb(DEGW), _rb(D)],
    out_shape=(
        jax.ShapeDtypeStruct((NPAD, DEGW), jnp.float32),
        jax.ShapeDtypeStruct((NPAD, DEGW), jnp.float32),
        jax.ShapeDtypeStruct((NPAD, D), jnp.float32),
    ),
)

_mid = pl.pallas_call(
    _mid_body,
    grid=(GRID,),
    in_specs=[_rb(D), _rb(D), _rb(D), _rb(DEGW), _rb(DEGW), _FULL_B, _FULL_W,
              _rb(D)],
    out_specs=[_rb(D), _rb(D)],
    out_shape=(
        jax.ShapeDtypeStruct((NPAD, D), jnp.float32),
        jax.ShapeDtypeStruct((NPAD, D), jnp.float32),
    ),
)

_fin = pl.pallas_call(
    _fin_body,
    grid=(GRID,),
    in_specs=[_rb(D), _rb(D), _rb(D), _rb(DEGW), _FULL_B, _rb(D)],
    out_specs=_rb(D),
    out_shape=jax.ShapeDtypeStruct((NPAD, D), jnp.float32),
)


def kernel(features, edge_index, W1, b1, W2, b2, W3, b3):
    ei = edge_index.astype(jnp.int32)
    pad_d = jnp.broadcast_to(N + jnp.arange(EPTP - EPT, dtype=jnp.int32),
                             (NW, EPTP - EPT))
    pad_s = pad_d
    srcs = jnp.concatenate([ei[0].reshape(NW, EPT), pad_s], axis=1)
    srcs = srcs.reshape(NW, NCH, CHUNK)
    dsts = jnp.concatenate([ei[1].reshape(NW, EPT), pad_d], axis=1)
    dsts = dsts.reshape(NW, NCH, CHUNK)

    feats = jnp.pad(features, ((0, NPAD - N), (0, 0)))
    zd = jnp.zeros((NPAD, D), jnp.float32)
    ones_t = jnp.ones((NPAD, D), jnp.float32)
    b1r = b1.reshape(1, D)
    b2r = b2.reshape(1, D)
    b3r = b3.reshape(1, D)

    seq = jnp.broadcast_to(
        (jnp.arange(EPTP, dtype=jnp.int32) % NPAD).reshape(1, NCH, CHUNK),
        (NW, NCH, CHUNK))
    deg_o = _agg_kernel(ones_t, seq, srcs, zd)
    deg_i = _agg_kernel(ones_t, seq, dsts, zd)
    ns, nd, msg1 = _prep(deg_o[0], deg_o[1], deg_i[0], deg_i[1], feats, W1)

    part1 = _agg_kernel(msg1, srcs, dsts, zd)
    jk1, msg2 = _mid(part1[0], part1[1], msg1, nd, ns, b1r, W2, zd)

    part2 = _agg_kernel(msg2, srcs, dsts, zd)
    jk2, msg3 = _mid(part2[0], part2[1], msg2, nd, ns, b2r, W3, jk1)

    part3 = _agg_kernel(msg3, srcs, dsts, zd)
    out = _fin(part3[0], part3[1], msg3, nd, b3r, jk2)
    return out[:N]


# speedup vs baseline: 1.0148x; 1.0148x over previous
import jax
import jax.numpy as jnp
from jax import lax
from jax.experimental import pallas as pl
from jax.experimental.pallas import tpu as pltpu
from jax.experimental.pallas import tpu_sc as plsc

N = 10000
D = 128
E = 320000

NC = 2
NS = 16
NW = NC * NS
NPAD = 10240
RPT = NPAD // NS
EPT = E // NW
CHUNK = 128
NCH = 79
EPTP = NCH * CHUNK
DEGW = 16

BR = 1280
GRID = NPAD // BR

_mesh = plsc.VectorSubcoreMesh(core_axis_name="c", subcore_axis_name="s",
                               num_cores=NC, num_subcores=NS)


_AGG_KERNEL_ARGS = dict(
    out_type=jax.ShapeDtypeStruct((NC, NPAD, D), jnp.float32),
    mesh=_mesh,
    scratch_types=[
        pltpu.VMEM((NCH, CHUNK), jnp.int32),
        pltpu.VMEM((NCH, CHUNK), jnp.int32),
        pltpu.VMEM((CHUNK, D), jnp.float32),
        pltpu.VMEM_SHARED((NPAD, D), jnp.float32),
        pltpu.SemaphoreType.DMA,
    ],
)


def _agg_body(msg, srcs, dsts, zd_h, part, si, di, rows, acc, gsem):
    c = lax.axis_index("c")
    s = lax.axis_index("s")
    wid = s * NC + c
    rs = pl.ds(s * RPT, RPT)
    pltpu.sync_copy(zd_h.at[rs], acc.at[rs])
    pltpu.sync_copy(srcs.at[wid], si)
    pltpu.sync_copy(dsts.at[wid], di)
    plsc.subcore_barrier()

    def body(j, carry):
        pltpu.async_copy(msg.at[si.at[j]], rows, gsem).wait()
        pltpu.sync_copy(rows, acc.at[di.at[j]], add=True)
        return carry

    lax.fori_loop(0, NCH, body, 0)
    plsc.subcore_barrier()
    pltpu.sync_copy(acc.at[rs], part.at[c, rs])


_agg_kernel = pl.kernel(_agg_body, **_AGG_KERNEL_ARGS)


def _prep_body(do0, do1, di0, di1, f, w, ns_o, nd_o, msg_o):
    ns = lax.rsqrt(do0[:, 0:1] + do1[:, 0:1] + 1.0)
    nd = lax.rsqrt(di0[:, 0:1] + di1[:, 0:1] + 1.0)
    wide = jnp.ones((1, DEGW), jnp.float32)
    ns_o[...] = ns * wide
    nd_o[...] = nd * wide
    msg_o[...] = jnp.dot(f[...] * ns, w[...],
                         preferred_element_type=jnp.float32)


def _mid_body(p0, p1, msg, nd, ns, b, w, jk_in, jk_o, msg_o):
    agg = p0[...] + p1[...] + msg[...]
    h = jnp.maximum(agg * nd[:, 0:1] + b[...], 0.0)
    jk_o[...] = jnp.maximum(jk_in[...], h)
    msg_o[...] = jnp.dot(h * ns[:, 0:1], w[...],
                         preferred_element_type=jnp.float32)


def _fin_body(p0, p1, msg, nd, b, jk_in, jk_o):
    agg = p0[...] + p1[...] + msg[...]
    h = jnp.maximum(agg * nd[:, 0:1] + b[...], 0.0)
    jk_o[...] = jnp.maximum(jk_in[...], h)


def _rb(width):
    return pl.BlockSpec((BR, width), lambda i: (i, 0))


_FULL_W = pl.BlockSpec((D, D), lambda i: (0, 0))
_FULL_B = pl.BlockSpec((1, D), lambda i: (0, 0))

_prep = pl.pallas_call(
    _prep_body,
    grid=(GRID,),
    in_specs=[_rb(D), _rb(D), _rb(D), _rb(D), _rb(D), _FULL_W],
    out_specs=[_rb(DEGW), _rb(DEGW), _rb(D)],
    out_shape=(
        jax.ShapeDtypeStruct((NPAD, DEGW), jnp.float32),
        jax.ShapeDtypeStruct((NPAD, DEGW), jnp.float32),
        jax.ShapeDtypeStruct((NPAD, D), jnp.float32),
    ),
)

_mid = pl.pallas_call(
    _mid_body,
    grid=(GRID,),
    in_specs=[_rb(D), _rb(D), _rb(D), _rb(DEGW), _rb(DEGW), _FULL_B, _FULL_W,
              _rb(D)],
    out_specs=[_rb(D), _rb(D)],
    out_shape=(
        jax.ShapeDtypeStruct((NPAD, D), jnp.float32),
        jax.ShapeDtypeStruct((NPAD, D), jnp.float32),
    ),
)

_fin = pl.pallas_call(
    _fin_body,
    grid=(GRID,),
    in_specs=[_rb(D), _rb(D), _rb(D), _rb(DEGW), _FULL_B, _rb(D)],
    out_specs=_rb(D),
    out_shape=jax.ShapeDtypeStruct((NPAD, D), jnp.float32),
)


def kernel(features, edge_index, W1, b1, W2, b2, W3, b3):
    ei = edge_index.astype(jnp.int32)
    pad_d = jnp.broadcast_to(N + jnp.arange(EPTP - EPT, dtype=jnp.int32),
                             (NW, EPTP - EPT))
    pad_s = pad_d
    srcs = jnp.concatenate([ei[0].reshape(NW, EPT), pad_s], axis=1)
    srcs = srcs.reshape(NW, NCH, CHUNK)
    dsts = jnp.concatenate([ei[1].reshape(NW, EPT), pad_d], axis=1)
    dsts = dsts.reshape(NW, NCH, CHUNK)

    feats = jnp.pad(features, ((0, NPAD - N), (0, 0)))
    zd = jnp.zeros((NPAD, D), jnp.float32)
    ones_t = jnp.ones((NPAD, D), jnp.float32)
    b1r = b1.reshape(1, D)
    b2r = b2.reshape(1, D)
    b3r = b3.reshape(1, D)

    deg_o = _agg_kernel(ones_t, srcs, srcs, zd)
    deg_i = _agg_kernel(ones_t, dsts, dsts, zd)
    ns, nd, msg1 = _prep(deg_o[0], deg_o[1], deg_i[0], deg_i[1], feats, W1)

    part1 = _agg_kernel(msg1, srcs, dsts, zd)
    jk1, msg2 = _mid(part1[0], part1[1], msg1, nd, ns, b1r, W2, zd)

    part2 = _agg_kernel(msg2, srcs, dsts, zd)
    jk2, msg3 = _mid(part2[0], part2[1], msg2, nd, ns, b2r, W3, jk1)

    part3 = _agg_kernel(msg3, srcs, dsts, zd)
    out = _fin(part3[0], part3[1], msg3, nd, b3r, jk2)
    return out[:N]
